# Initial kernel scaffold; baseline (speedup 1.0000x reference)
import jax
import jax.numpy as jnp
from jax.experimental import pallas as pl


def kernel(input, weight):
    raise NotImplementedError("write your pallas kernel here")


# speedup vs baseline: 1.0234x; 1.0234x over previous
import functools

import jax
import jax.numpy as jnp
from jax import lax
from jax.experimental import pallas as pl
from jax.experimental.pallas import tpu as pltpu
from jax.experimental.pallas import tpu_sc as plsc

NUM_EMB = 1000000
DIM = 32
BATCH = 16384
HIST = 50

NC = 2
NS = 16
NW = NC * NS

TOTAL = BATCH * HIST
PER_W = TOTAL // NW
CHUNK = 128
NCHUNK = PER_W // CHUNK


@functools.partial(
    pl.kernel,
    out_type=jax.ShapeDtypeStruct((TOTAL, DIM), jnp.float32),
    mesh=plsc.VectorSubcoreMesh(core_axis_name="c", subcore_axis_name="s"),
    compiler_params=pltpu.CompilerParams(use_tc_tiling_on_sc=False),
    scratch_types=[
        pltpu.VMEM((NCHUNK, CHUNK), jnp.int32),
        pltpu.VMEM((CHUNK, DIM), jnp.float32),
        pltpu.SemaphoreType.DMA,
    ],
)
def _emb_lookup(idx_hbm, table_hbm, out_hbm, idx_v, rows_v, sem):
    wid = lax.axis_index("s") * NC + lax.axis_index("c")
    base = wid * PER_W
    pltpu.sync_copy(idx_hbm.at[wid], idx_v)

    def body(j, carry):
        pltpu.async_copy(table_hbm.at[idx_v.at[j]], rows_v, sem).wait()
        pltpu.sync_copy(rows_v, out_hbm.at[pl.ds(base + j * CHUNK, CHUNK)])
        return carry

    lax.fori_loop(0, NCHUNK, body, 0)


def kernel(input, weight):
    idx = input.reshape(NW, NCHUNK, CHUNK)
    out = _emb_lookup(idx, weight)
    return out.reshape(BATCH, HIST, DIM)


# speedup vs baseline: 1.1106x; 1.0852x over previous
import functools

import jax
import jax.numpy as jnp
from jax import lax
from jax.experimental import pallas as pl
from jax.experimental.pallas import tpu as pltpu
from jax.experimental.pallas import tpu_sc as plsc

NUM_EMB = 1000000
DIM = 32
BATCH = 16384
HIST = 50

NC = 2
NS = 16
NW = NC * NS

TOTAL = BATCH * HIST
PER_W = TOTAL // NW
CHUNK = 128
NCHUNK = PER_W // CHUNK
G = 10
NPAIR = NCHUNK // (2 * G)


@functools.partial(
    pl.kernel,
    out_type=jax.ShapeDtypeStruct((TOTAL, DIM), jnp.float32),
    mesh=plsc.VectorSubcoreMesh(core_axis_name="c", subcore_axis_name="s"),
    compiler_params=pltpu.CompilerParams(use_tc_tiling_on_sc=False),
    scratch_types=[
        pltpu.VMEM((NCHUNK, CHUNK), jnp.int32),
        pltpu.VMEM((2, G, CHUNK, DIM), jnp.float32),
        pltpu.SemaphoreType.DMA,
        pltpu.SemaphoreType.DMA,
        pltpu.SemaphoreType.DMA,
        pltpu.SemaphoreType.DMA,
    ],
)
def _emb_lookup(idx_hbm, table_hbm, out_hbm, idx_v, rows_v, sg0, sg1, ss0, ss1):
    wid = lax.axis_index("s") * NC + lax.axis_index("c")
    base = wid * PER_W
    pltpu.sync_copy(idx_hbm.at[wid], idx_v)
    sem_g = (sg0, sg1)
    sem_s = (ss0, ss1)

    def out_slice(chunk):
        return out_hbm.at[pl.ds(base + chunk * CHUNK, CHUNK)]

    def body(p, carry):
        for s in range(2):
            g0 = p * 2 * G + s * G

            @pl.when(p > 0)
            def _():
                for b in range(G):
                    pltpu.make_async_copy(
                        rows_v.at[s, b], out_slice(g0 + b), sem_s[s]
                    ).wait()

            for b in range(G):
                pltpu.make_async_copy(
                    table_hbm.at[idx_v.at[g0 + b]], rows_v.at[s, b], sem_g[s]
                ).start()
            for b in range(G):
                pltpu.make_async_copy(
                    table_hbm.at[idx_v.at[g0 + b]], rows_v.at[s, b], sem_g[s]
                ).wait()
            for b in range(G):
                pltpu.make_async_copy(
                    rows_v.at[s, b], out_slice(g0 + b), sem_s[s]
                ).start()
        return carry

    lax.fori_loop(0, NPAIR, body, 0)

    last = (NPAIR - 1) * 2 * G
    for s in range(2):
        for b in range(G):
            pltpu.make_async_copy(
                rows_v.at[s, b], out_slice(last + s * G + b), sem_s[s]
            ).wait()


def kernel(input, weight):
    idx = input.reshape(NW, NCHUNK, CHUNK)
    out = _emb_lookup(idx, weight)
    return out.reshape(BATCH, HIST, DIM)


# speedup vs baseline: 1.5420x; 1.3885x over previous
import functools

import jax
import jax.numpy as jnp
from jax import lax
from jax.experimental import pallas as pl
from jax.experimental.pallas import tpu as pltpu
from jax.experimental.pallas import tpu_sc as plsc

NUM_EMB = 1000000
DIM = 32
BATCH = 16384
HIST = 50

NC = 2
NS = 16
NW = NC * NS

BT = 128
NBT = BATCH // BT
BT_PER_W = NBT // NW
NCHUNK = BT_PER_W * HIST


@functools.partial(
    pl.kernel,
    out_type=jax.ShapeDtypeStruct((HIST, DIM // 8, NBT, 8, BT), jnp.float32),
    mesh=plsc.VectorSubcoreMesh(core_axis_name="c", subcore_axis_name="s"),
    compiler_params=pltpu.CompilerParams(
        use_tc_tiling_on_sc=False, needs_layout_passes=False
    ),
    scratch_types=[
        pltpu.VMEM((HIST, BT_PER_W * BT), jnp.int32),
        pltpu.VMEM((2, BT, DIM), jnp.float32),
        pltpu.VMEM((2, DIM // 8, 8, BT), jnp.float32),
        pltpu.SemaphoreType.DMA,
        pltpu.SemaphoreType.DMA,
        pltpu.SemaphoreType.DMA,
        pltpu.SemaphoreType.DMA,
    ],
)
def _emb_lookup(idxT_hbm, table_hbm, out_hbm, idx_v, lbuf, obuf,
                sg0, sg1, ss0, ss1):
    wid = lax.axis_index("s") * NC + lax.axis_index("c")
    bt0 = wid * BT_PER_W
    pltpu.sync_copy(idxT_hbm.at[:, pl.ds(bt0 * BT, BT_PER_W * BT)], idx_v)
    sem_g = (sg0, sg1)
    sem_s = (ss0, ss1)

    def gather_desc(c, buf):
        btl = c // HIST
        h = c - btl * HIST
        return pltpu.make_async_copy(
            table_hbm.at[idx_v.at[h, pl.ds(btl * BT, BT)]],
            lbuf.at[buf],
            sem_g[buf],
        )

    def scatter_desc(c, buf):
        btl = c // HIST
        h = c - btl * HIST
        return pltpu.make_async_copy(
            obuf.at[buf], out_hbm.at[h, :, bt0 + btl, :, :], sem_s[buf]
        )

    def process(c, buf):
        @pl.when(c + 1 < NCHUNK)
        def _():
            gather_desc(c + 1, 1 - buf).start()

        gather_desc(c, buf).wait()

        @pl.when(c >= 2)
        def _():
            scatter_desc(c, buf).wait()

        for kg in range(BT // 16):
            rowv = lax.iota(jnp.int32, 16) + 16 * kg
            for d in range(DIM):
                colv = jnp.full((16,), d, jnp.int32)
                vals = plsc.load_gather(lbuf.at[buf], [rowv, colv])
                obuf[buf, d // 8, d % 8, pl.ds(16 * kg, 16)] = vals

        scatter_desc(c, buf).start()

    gather_desc(0, 0).start()

    def body(i, carry):
        process(2 * i, 0)
        process(2 * i + 1, 1)
        return carry

    lax.fori_loop(0, NCHUNK // 2, body, 0)

    scatter_desc(NCHUNK - 2, 0).wait()
    scatter_desc(NCHUNK - 1, 1).wait()


def kernel(input, weight):
    out5 = _emb_lookup(input.T, weight)
    return out5.transpose(2, 4, 0, 1, 3).reshape(BATCH, HIST, DIM)


# speedup vs baseline: 1.9708x; 1.2781x over previous
import functools

import jax
import jax.numpy as jnp
from jax import lax
from jax.experimental import pallas as pl
from jax.experimental.pallas import tpu as pltpu
from jax.experimental.pallas import tpu_sc as plsc

NUM_EMB = 1000000
DIM = 32
BATCH = 16384
HIST = 50

NC = 2
NS = 16
NW = NC * NS

BT = 128
NBT = BATCH // BT
BT_PER_W = NBT // NW
NCHUNK = BT_PER_W * HIST
NLINE = NUM_EMB // 4


@functools.partial(
    pl.kernel,
    out_type=jax.ShapeDtypeStruct((HIST, DIM // 8, NBT, 8, BT), jnp.float32),
    mesh=plsc.VectorSubcoreMesh(core_axis_name="c", subcore_axis_name="s"),
    compiler_params=pltpu.CompilerParams(
        use_tc_tiling_on_sc=False, needs_layout_passes=False
    ),
    scratch_types=[
        pltpu.VMEM((HIST, BT_PER_W * BT), jnp.int32),
        pltpu.VMEM((2, BT), jnp.int32),
        pltpu.VMEM((2, BT), jnp.int32),
        pltpu.VMEM((2, BT, 128), jnp.float32),
        pltpu.VMEM((2, DIM // 8, 8, BT), jnp.float32),
        pltpu.SemaphoreType.DMA,
        pltpu.SemaphoreType.DMA,
        pltpu.SemaphoreType.DMA,
        pltpu.SemaphoreType.DMA,
    ],
)
def _emb_lookup(idxT_hbm, lines_hbm, out_hbm, idx_v, lines_v, subs_v,
                lbuf, obuf, sg0, sg1, ss0, ss1):
    wid = lax.axis_index("s") * NC + lax.axis_index("c")
    bt0 = wid * BT_PER_W
    pltpu.sync_copy(idxT_hbm.at[:, pl.ds(bt0 * BT, BT_PER_W * BT)], idx_v)
    sem_g = (sg0, sg1)
    sem_s = (ss0, ss1)
    iota = lax.iota(jnp.int32, 16)

    def chunk_hb(c):
        btl = c // HIST
        return btl, c - btl * HIST

    def prep_and_fire(c, buf):
        btl, h = chunk_hb(c)
        for j in range(BT // 16):
            r = idx_v[h, pl.ds(btl * BT + 16 * j, 16)]
            lines_v[buf, pl.ds(16 * j, 16)] = lax.shift_right_logical(r, 2)
            subs_v[buf, pl.ds(16 * j, 16)] = lax.shift_left(
                lax.bitwise_and(r, 3), 5)
        pltpu.make_async_copy(
            lines_hbm.at[lines_v.at[buf]], lbuf.at[buf], sem_g[buf]
        ).start()

    def wait_gather(c, buf):
        pltpu.make_async_copy(
            lines_hbm.at[lines_v.at[buf]], lbuf.at[buf], sem_g[buf]
        ).wait()

    def scatter_desc(c, buf):
        btl, h = chunk_hb(c)
        return pltpu.make_async_copy(
            obuf.at[buf], out_hbm.at[h, :, bt0 + btl, :, :], sem_s[buf]
        )

    def process(c, buf):
        @pl.when(c + 1 < NCHUNK)
        def _():
            prep_and_fire(c + 1, 1 - buf)

        wait_gather(c, buf)

        @pl.when(c >= 2)
        def _():
            scatter_desc(c, buf).wait()

        for kg in range(BT // 16):
            rowv = iota + 16 * kg
            subv = subs_v[buf, pl.ds(16 * kg, 16)]
            for p in range(DIM):
                dsw = lax.bitwise_and(iota + p, 31)
                colv = subv + dsw
                vals = plsc.load_gather(lbuf.at[buf], [rowv, colv])
                plsc.store_scatter(
                    obuf.at[buf],
                    [lax.shift_right_logical(dsw, 3),
                     lax.bitwise_and(dsw, 7), rowv],
                    vals,
                )

        scatter_desc(c, buf).start()

    prep_and_fire(0, 0)

    def body(i, carry):
        process(2 * i, 0)
        process(2 * i + 1, 1)
        return carry

    lax.fori_loop(0, NCHUNK // 2, body, 0)

    scatter_desc(NCHUNK - 2, 0).wait()
    scatter_desc(NCHUNK - 1, 1).wait()


def kernel(input, weight):
    out5 = _emb_lookup(input.T, weight.reshape(NLINE, 128))
    return out5.transpose(2, 4, 0, 1, 3).reshape(BATCH, HIST, DIM)


# speedup vs baseline: 1.9992x; 1.0144x over previous
import functools

import jax
import jax.numpy as jnp
from jax import lax
from jax.experimental import pallas as pl
from jax.experimental.pallas import tpu as pltpu
from jax.experimental.pallas import tpu_sc as plsc

NUM_EMB = 1000000
DIM = 32
BATCH = 16384
HIST = 50

RELAYOUT_W = 2048

NC = 2
NS = 16
NW = NC * NS

BT = 128
NBT = BATCH // BT
BT_PER_W = NBT // NW
NCHUNK = BT_PER_W * HIST
NLINE = NUM_EMB // 4


@functools.partial(
    pl.kernel,
    out_type=jax.ShapeDtypeStruct((HIST, DIM // 8, NBT, 8, BT), jnp.float32),
    mesh=plsc.VectorSubcoreMesh(core_axis_name="c", subcore_axis_name="s"),
    compiler_params=pltpu.CompilerParams(
        use_tc_tiling_on_sc=False, needs_layout_passes=False
    ),
    scratch_types=[
        pltpu.VMEM((HIST, BT_PER_W * BT), jnp.int32),
        pltpu.VMEM((2, BT), jnp.int32),
        pltpu.VMEM((2, BT), jnp.int32),
        pltpu.VMEM((2, BT, 128), jnp.float32),
        pltpu.VMEM((2, DIM // 8, 8, BT), jnp.float32),
        pltpu.SemaphoreType.DMA,
        pltpu.SemaphoreType.DMA,
        pltpu.SemaphoreType.DMA,
        pltpu.SemaphoreType.DMA,
    ],
)
def _emb_lookup(idxT_hbm, lines_hbm, out_hbm, idx_v, lines_v, subs_v,
                lbuf, obuf, sg0, sg1, ss0, ss1):
    wid = lax.axis_index("s") * NC + lax.axis_index("c")
    bt0 = wid * BT_PER_W
    pltpu.sync_copy(idxT_hbm.at[:, pl.ds(bt0 * BT, BT_PER_W * BT)], idx_v)
    sem_g = (sg0, sg1)
    sem_s = (ss0, ss1)
    iota = lax.iota(jnp.int32, 16)

    def chunk_hb(c):
        btl = c // HIST
        return btl, c - btl * HIST

    def prep_and_fire(c, buf):
        btl, h = chunk_hb(c)
        for j in range(BT // 16):
            r = idx_v[h, pl.ds(btl * BT + 16 * j, 16)]
            lines_v[buf, pl.ds(16 * j, 16)] = lax.shift_right_logical(r, 2)
            subs_v[buf, pl.ds(16 * j, 16)] = lax.shift_left(
                lax.bitwise_and(r, 3), 5)
        pltpu.make_async_copy(
            lines_hbm.at[lines_v.at[buf]], lbuf.at[buf], sem_g[buf]
        ).start()

    def wait_gather(c, buf):
        pltpu.make_async_copy(
            lines_hbm.at[lines_v.at[buf]], lbuf.at[buf], sem_g[buf]
        ).wait()

    def scatter_desc(c, buf):
        btl, h = chunk_hb(c)
        return pltpu.make_async_copy(
            obuf.at[buf], out_hbm.at[h, :, bt0 + btl, :, :], sem_s[buf]
        )

    def process(c, buf):
        @pl.when(c + 1 < NCHUNK)
        def _():
            prep_and_fire(c + 1, 1 - buf)

        wait_gather(c, buf)

        @pl.when(c >= 2)
        def _():
            scatter_desc(c, buf).wait()

        for kg in range(BT // 16):
            rowv = iota + 16 * kg
            subv = subs_v[buf, pl.ds(16 * kg, 16)]
            for p in range(DIM):
                dsw = lax.bitwise_and(iota + p, 31)
                colv = subv + dsw
                vals = plsc.load_gather(lbuf.at[buf], [rowv, colv])
                plsc.store_scatter(
                    obuf.at[buf],
                    [lax.shift_right_logical(dsw, 3),
                     lax.bitwise_and(dsw, 7), rowv],
                    vals,
                )

        scatter_desc(c, buf).start()

    prep_and_fire(0, 0)

    def body(i, carry):
        process(2 * i, 0)
        process(2 * i + 1, 1)
        return carry

    lax.fori_loop(0, NCHUNK // 2, body, 0)

    scatter_desc(NCHUNK - 2, 0).wait()
    scatter_desc(NCHUNK - 1, 1).wait()


def _relayout_body(wt_ref, out_ref):
    t = wt_ref[...].T
    t3 = t.reshape(RELAYOUT_W // 4, 4, DIM)
    out_ref[...] = jnp.concatenate([t3[:, s, :] for s in range(4)], axis=1)


_relayout = pl.pallas_call(
    _relayout_body,
    out_shape=jax.ShapeDtypeStruct((NLINE, 128), jnp.float32),
    grid=((NUM_EMB + RELAYOUT_W - 1) // RELAYOUT_W,),
    in_specs=[pl.BlockSpec((DIM, RELAYOUT_W), lambda i: (0, i))],
    out_specs=pl.BlockSpec((RELAYOUT_W // 4, 128), lambda i: (i, 0)),
)


def kernel(input, weight):
    w128 = _relayout(weight.T)
    out5 = _emb_lookup(input.T, w128)
    return out5.transpose(2, 4, 0, 1, 3).reshape(BATCH, HIST, DIM)
